# baseline (speedup 1.0000x reference)
import functools

import jax
import jax.numpy as jnp
from jax import lax
from jax.experimental import pallas as pl
from jax.experimental.pallas import tpu as pltpu
from jax.experimental.pallas import tpu_sc as plsc

N_NODES = 10000
N_EDGES = 320000
D = 128
BATCH = 2000

NC = 2
NS = 16
NW = NC * NS
EPW = N_EDGES // NW
CH = 80
NCHUNK = EPW // CH
SUB_ROWS = 624
TAIL0 = NS * SUB_ROWS
TAIL_ROWS = N_NODES - TAIL0


def _sc_agg_body(with_cnt, h_hbm, src_hbm, dst_hbm, z2_hbm, *rest):
    if with_cnt:
        acc_out, cnt_out = rest[0], rest[1]
        rest = rest[2:]
    else:
        acc_out = rest[0]
        rest = rest[1:]
    (src_all, dstv0, dstv1, rows0, rows1, onesv, zv, acc_sh, cnt_sh,
     sem_g0, sem_g1, sem_s0, sem_s1, sem_d0, sem_d1, sem_c0, sem_c1) = rest
    dstv = (dstv0, dstv1)
    rowsv = (rows0, rows1)
    sem_g = (sem_g0, sem_g1)
    sem_s = (sem_s0, sem_s1)
    sem_d = (sem_d0, sem_d1)
    sem_c = (sem_c0, sem_c1)

    c = lax.axis_index("c")
    s = lax.axis_index("s")
    r0 = pl.multiple_of(s * SUB_ROWS, 8)
    rows = pl.ds(r0, SUB_ROWS)
    tail = pl.ds(TAIL0, TAIL_ROWS)

    pltpu.sync_copy(z2_hbm.at[rows], acc_sh.at[rows])

    @pl.when(s == 0)
    def _zero_tail():
        pltpu.sync_copy(z2_hbm.at[tail], acc_sh.at[tail])

    if with_cnt:
        for j in range(SUB_ROWS // 16):
            zv[pl.ds(j * 16, 16)] = jnp.zeros((16,), jnp.float32)
        pltpu.sync_copy(zv, cnt_sh.at[rows])

        @pl.when(s == 0)
        def _zero_cnt_tail():
            pltpu.sync_copy(zv.at[pl.ds(0, TAIL_ROWS)], cnt_sh.at[tail])

        for j in range(CH // 16):
            onesv[pl.ds(j * 16, 16)] = jnp.full((16,), 1.0, jnp.float32)
    plsc.subcore_barrier()

    ebase = (c * NS + s) * EPW

    pltpu.sync_copy(src_hbm.at[pl.ds(ebase, EPW)], src_all)

    def slot(k, b):
        @pl.when(jnp.logical_and(k >= 2, k < NCHUNK + 2))
        def _wait_scatter():
            pltpu.make_async_copy(rowsv[b], acc_sh.at[dstv[b]], sem_s[b]).wait()
            if with_cnt:
                pltpu.make_async_copy(onesv, cnt_sh.at[dstv[b]], sem_c[b]).wait()

        @pl.when(k < NCHUNK)
        def _issue():
            eoff = pl.multiple_of(ebase + k * CH, 8)
            pltpu.async_copy(dst_hbm.at[pl.ds(eoff, CH)], dstv[b], sem_d[b])
            coff = pl.multiple_of(k * CH, 8)
            pltpu.async_copy(h_hbm.at[src_all.at[pl.ds(coff, CH)]], rowsv[b],
                             sem_g[b])

        @pl.when(jnp.logical_and(k >= 1, k < NCHUNK + 1))
        def _scatter_prev():
            p = 1 - b
            pltpu.make_async_copy(
                h_hbm.at[src_all.at[pl.ds(0, CH)]], rowsv[p], sem_g[p]).wait()
            pltpu.make_async_copy(
                dst_hbm.at[pl.ds(0, CH)], dstv[p], sem_d[p]).wait()
            pltpu.async_copy(rowsv[p], acc_sh.at[dstv[p]], sem_s[p], add=True)
            if with_cnt:
                pltpu.async_copy(onesv, cnt_sh.at[dstv[p]], sem_c[p], add=True)

    def pair(kk, carry):
        k0 = kk * 2
        slot(k0, 0)
        slot(k0 + 1, 1)
        return carry

    lax.fori_loop(0, (NCHUNK + 2 + 1) // 2, pair, 0)
    plsc.subcore_barrier()

    pltpu.sync_copy(acc_sh.at[rows], acc_out.at[c, rows])

    @pl.when(s == 0)
    def _out_tail():
        pltpu.sync_copy(acc_sh.at[tail], acc_out.at[c, tail])

    if with_cnt:
        cbase = c * N_NODES
        pltpu.sync_copy(cnt_sh.at[rows], zv)
        pltpu.sync_copy(zv, cnt_out.at[pl.ds(pl.multiple_of(cbase + r0, 8),
                                             SUB_ROWS)])

        @pl.when(s == 0)
        def _out_cnt_tail():
            pltpu.sync_copy(cnt_sh.at[tail], zv.at[pl.ds(0, TAIL_ROWS)])
            pltpu.sync_copy(
                zv.at[pl.ds(0, TAIL_ROWS)],
                cnt_out.at[pl.ds(pl.multiple_of(cbase + TAIL0, 8), TAIL_ROWS)])


def _make_sc_agg(with_cnt):
    mesh = plsc.VectorSubcoreMesh(core_axis_name="c", subcore_axis_name="s")
    out_type = [jax.ShapeDtypeStruct((NC, N_NODES, D), jnp.float32)]
    if with_cnt:
        out_type.append(jax.ShapeDtypeStruct((NC * N_NODES,), jnp.float32))
    scratch = [
        pltpu.VMEM((EPW,), jnp.int32),
        pltpu.VMEM((CH,), jnp.int32),
        pltpu.VMEM((CH,), jnp.int32),
        pltpu.VMEM((CH, D), jnp.float32),
        pltpu.VMEM((CH, D), jnp.float32),
        pltpu.VMEM((CH,), jnp.float32),
        pltpu.VMEM((SUB_ROWS,), jnp.float32),
        pltpu.VMEM_SHARED((N_NODES, D), jnp.float32),
        pltpu.VMEM_SHARED((N_NODES,), jnp.float32),
    ] + [pltpu.SemaphoreType.DMA] * 8
    return pl.kernel(
        functools.partial(_sc_agg_body, with_cnt),
        out_type=out_type,
        mesh=mesh,
        scratch_types=scratch,
    )


_sc_agg_cnt = _make_sc_agg(True)
_sc_agg = _make_sc_agg(False)


def _tc_pre_body(x_ref, wr_ref, bl_ref, o_ref):
    o_ref[...] = (
        jnp.dot(x_ref[...], wr_ref[...], preferred_element_type=jnp.float32)
        + bl_ref[...]
    )


def _tc_pre(x, Wr, bl, block_rows=2000):
    R = block_rows
    return pl.pallas_call(
        _tc_pre_body,
        grid=(N_NODES // R,),
        in_specs=[
            pl.BlockSpec((R, D), lambda i: (i, 0)),
            pl.BlockSpec((D, D), lambda i: (0, 0)),
            pl.BlockSpec((1, D), lambda i: (0, 0)),
        ],
        out_specs=pl.BlockSpec((R, D), lambda i: (i, 0)),
        out_shape=jax.ShapeDtypeStruct((N_NODES, D), jnp.float32),
    )(x, Wr, bl.reshape(1, D))


def _tc_post_body(acc_ref, cnt_ref, p_ref, wl_ref, o_ref):
    a = acc_ref[0] + acc_ref[1]
    inv = 1.0 / jnp.maximum(cnt_ref[0] + cnt_ref[1], 1.0)
    mean = a * inv
    o_ref[...] = jnp.maximum(
        jnp.dot(mean, wl_ref[...], preferred_element_type=jnp.float32)
        + p_ref[...],
        0.0,
    )


def _tc_post(acc, cnt3, p, Wl, block_rows=2000):
    R = block_rows
    return pl.pallas_call(
        _tc_post_body,
        grid=(N_NODES // R,),
        in_specs=[
            pl.BlockSpec((2, R, D), lambda i: (0, i, 0)),
            pl.BlockSpec((2, R, 1), lambda i: (0, i, 0)),
            pl.BlockSpec((R, D), lambda i: (i, 0)),
            pl.BlockSpec((D, D), lambda i: (0, 0)),
        ],
        out_specs=pl.BlockSpec((R, D), lambda i: (i, 0)),
        out_shape=jax.ShapeDtypeStruct((N_NODES, D), jnp.float32),
    )(acc, cnt3, p, Wl)


def _tc_fc_body(h_ref, w_ref, b_ref, o_ref):
    o_ref[...] = (
        jnp.dot(h_ref[...], w_ref[...], preferred_element_type=jnp.float32)
        + b_ref[...]
    )


def _tc_fc(h2, Wfc, bfc, block_rows=1000):
    R = block_rows
    K = 5 * D
    return pl.pallas_call(
        _tc_fc_body,
        grid=(BATCH // R,),
        in_specs=[
            pl.BlockSpec((R, K), lambda i: (i, 0)),
            pl.BlockSpec((K, D), lambda i: (0, 0)),
            pl.BlockSpec((1, D), lambda i: (0, 0)),
        ],
        out_specs=pl.BlockSpec((R, D), lambda i: (i, 0)),
        out_shape=jax.ShapeDtypeStruct((BATCH, D), jnp.float32),
    )(h2, Wfc, bfc.reshape(1, D))


def kernel(x, edge_index, Wl1, bl1, Wr1, Wl2, bl2, Wr2, Wl3, bl3, Wr3,
           Wl4, bl4, Wr4, Wl5, bl5, Wr5, Wfc, bfc):
    src = edge_index[0].astype(jnp.int32)
    dst = edge_index[1].astype(jnp.int32)
    z2 = jnp.zeros((N_NODES, D), jnp.float32)

    acc, cnt = _sc_agg_cnt(x, src, dst, z2)
    p = _tc_pre(x, Wr1, bl1)
    cnt3 = cnt.reshape(NC, N_NODES, 1)
    h = _tc_post(acc, cnt3, p, Wl1)
    for Wl, bl, Wr in ((Wl2, bl2, Wr2), (Wl3, bl3, Wr3),
                       (Wl4, bl4, Wr4), (Wl5, bl5, Wr5)):
        (acc,) = _sc_agg(h, src, dst, z2)
        p = _tc_pre(h, Wr, bl)
        h = _tc_post(acc, cnt3, p, Wl)

    return _tc_fc(h.reshape(BATCH, 5 * D), Wfc, bfc)


# speedup vs baseline: 1.1510x; 1.1510x over previous
import functools

import jax
import jax.numpy as jnp
from jax import lax
from jax.experimental import pallas as pl
from jax.experimental.pallas import tpu as pltpu
from jax.experimental.pallas import tpu_sc as plsc

N_NODES = 10000
N_EDGES = 320000
D = 128
BATCH = 2000

NC = 2
NS = 16
NW = NC * NS
EPW = N_EDGES // NW
CH = 80
NCHUNK = EPW // CH
NBUF = 3
SUB_ROWS = 624
TAIL0 = NS * SUB_ROWS
TAIL_ROWS = N_NODES - TAIL0


def _sc_agg_body(with_cnt, h_hbm, src_hbm, dst_hbm, z2_hbm, *rest):
    if with_cnt:
        acc_out, cnt_out = rest[0], rest[1]
        rest = rest[2:]
    else:
        acc_out = rest[0]
        rest = rest[1:]
    src_all = rest[0]
    dstv = rest[1:1 + NBUF]
    rowsv = rest[1 + NBUF:1 + 2 * NBUF]
    onesv, zv, acc_sh, cnt_sh = rest[1 + 2 * NBUF:5 + 2 * NBUF]
    sems = rest[5 + 2 * NBUF:]
    sem_g = sems[0:NBUF]
    sem_s = sems[NBUF:2 * NBUF]
    sem_d = sems[2 * NBUF:3 * NBUF]
    sem_c = sems[3 * NBUF:4 * NBUF]

    c = lax.axis_index("c")
    s = lax.axis_index("s")
    r0 = pl.multiple_of(s * SUB_ROWS, 8)
    rows = pl.ds(r0, SUB_ROWS)
    tail = pl.ds(TAIL0, TAIL_ROWS)

    pltpu.sync_copy(z2_hbm.at[rows], acc_sh.at[rows])

    @pl.when(s == 0)
    def _zero_tail():
        pltpu.sync_copy(z2_hbm.at[tail], acc_sh.at[tail])

    if with_cnt:
        for j in range(SUB_ROWS // 16):
            zv[pl.ds(j * 16, 16)] = jnp.zeros((16,), jnp.float32)
        pltpu.sync_copy(zv, cnt_sh.at[rows])

        @pl.when(s == 0)
        def _zero_cnt_tail():
            pltpu.sync_copy(zv.at[pl.ds(0, TAIL_ROWS)], cnt_sh.at[tail])

        for j in range(CH // 16):
            onesv[pl.ds(j * 16, 16)] = jnp.full((16,), 1.0, jnp.float32)
    plsc.subcore_barrier()

    ebase = (c * NS + s) * EPW

    pltpu.sync_copy(src_hbm.at[pl.ds(ebase, EPW)], src_all)

    def slot(k, b):
        @pl.when(jnp.logical_and(k >= NBUF, k < NCHUNK + NBUF))
        def _wait_scatter():
            pltpu.make_async_copy(rowsv[b], acc_sh.at[dstv[b]], sem_s[b]).wait()
            if with_cnt:
                pltpu.make_async_copy(onesv, cnt_sh.at[dstv[b]], sem_c[b]).wait()

        @pl.when(k < NCHUNK)
        def _issue():
            eoff = pl.multiple_of(ebase + k * CH, 8)
            pltpu.async_copy(dst_hbm.at[pl.ds(eoff, CH)], dstv[b], sem_d[b])
            coff = pl.multiple_of(k * CH, 8)
            pltpu.async_copy(h_hbm.at[src_all.at[pl.ds(coff, CH)]], rowsv[b],
                             sem_g[b])

        @pl.when(jnp.logical_and(k >= 1, k < NCHUNK + 1))
        def _scatter_prev():
            p = (b + NBUF - 1) % NBUF
            pltpu.make_async_copy(
                h_hbm.at[src_all.at[pl.ds(0, CH)]], rowsv[p], sem_g[p]).wait()
            pltpu.make_async_copy(
                dst_hbm.at[pl.ds(0, CH)], dstv[p], sem_d[p]).wait()
            pltpu.async_copy(rowsv[p], acc_sh.at[dstv[p]], sem_s[p], add=True)
            if with_cnt:
                pltpu.async_copy(onesv, cnt_sh.at[dstv[p]], sem_c[p], add=True)

    NSLOT = NCHUNK + NBUF
    NITER = (NSLOT + NBUF - 1) // NBUF

    def group(kk, carry):
        k0 = kk * NBUF
        for b in range(NBUF):
            slot(k0 + b, b)
        return carry

    lax.fori_loop(0, NITER, group, 0)
    plsc.subcore_barrier()

    pltpu.sync_copy(acc_sh.at[rows], acc_out.at[c, rows])

    @pl.when(s == 0)
    def _out_tail():
        pltpu.sync_copy(acc_sh.at[tail], acc_out.at[c, tail])

    if with_cnt:
        cbase = c * N_NODES
        pltpu.sync_copy(cnt_sh.at[rows], zv)
        pltpu.sync_copy(zv, cnt_out.at[pl.ds(pl.multiple_of(cbase + r0, 8),
                                             SUB_ROWS)])

        @pl.when(s == 0)
        def _out_cnt_tail():
            pltpu.sync_copy(cnt_sh.at[tail], zv.at[pl.ds(0, TAIL_ROWS)])
            pltpu.sync_copy(
                zv.at[pl.ds(0, TAIL_ROWS)],
                cnt_out.at[pl.ds(pl.multiple_of(cbase + TAIL0, 8), TAIL_ROWS)])


def _make_sc_agg(with_cnt):
    mesh = plsc.VectorSubcoreMesh(core_axis_name="c", subcore_axis_name="s")
    out_type = [jax.ShapeDtypeStruct((NC, N_NODES, D), jnp.float32)]
    if with_cnt:
        out_type.append(jax.ShapeDtypeStruct((NC * N_NODES,), jnp.float32))
    scratch = (
        [pltpu.VMEM((EPW,), jnp.int32)]
        + [pltpu.VMEM((CH,), jnp.int32)] * NBUF
        + [pltpu.VMEM((CH, D), jnp.float32)] * NBUF
        + [
            pltpu.VMEM((CH,), jnp.float32),
            pltpu.VMEM((SUB_ROWS,), jnp.float32),
            pltpu.VMEM_SHARED((N_NODES, D), jnp.float32),
            pltpu.VMEM_SHARED((N_NODES,), jnp.float32),
        ]
        + [pltpu.SemaphoreType.DMA] * (4 * NBUF)
    )
    return pl.kernel(
        functools.partial(_sc_agg_body, with_cnt),
        out_type=out_type,
        mesh=mesh,
        scratch_types=scratch,
    )


_sc_agg_cnt = _make_sc_agg(True)
_sc_agg = _make_sc_agg(False)


def _tc_pre_body(x_ref, wr_ref, bl_ref, o_ref):
    o_ref[...] = (
        jnp.dot(x_ref[...], wr_ref[...], preferred_element_type=jnp.float32)
        + bl_ref[...]
    )


def _tc_pre(x, Wr, bl, block_rows=2000):
    R = block_rows
    return pl.pallas_call(
        _tc_pre_body,
        grid=(N_NODES // R,),
        in_specs=[
            pl.BlockSpec((R, D), lambda i: (i, 0)),
            pl.BlockSpec((D, D), lambda i: (0, 0)),
            pl.BlockSpec((1, D), lambda i: (0, 0)),
        ],
        out_specs=pl.BlockSpec((R, D), lambda i: (i, 0)),
        out_shape=jax.ShapeDtypeStruct((N_NODES, D), jnp.float32),
    )(x, Wr, bl.reshape(1, D))


def _tc_post_body(acc_ref, cnt_ref, p_ref, wl_ref, o_ref):
    a = acc_ref[0] + acc_ref[1]
    inv = 1.0 / jnp.maximum(cnt_ref[0] + cnt_ref[1], 1.0)
    mean = a * inv
    o_ref[...] = jnp.maximum(
        jnp.dot(mean, wl_ref[...], preferred_element_type=jnp.float32)
        + p_ref[...],
        0.0,
    )


def _tc_post(acc, cnt3, p, Wl, block_rows=2000):
    R = block_rows
    return pl.pallas_call(
        _tc_post_body,
        grid=(N_NODES // R,),
        in_specs=[
            pl.BlockSpec((2, R, D), lambda i: (0, i, 0)),
            pl.BlockSpec((2, R, 1), lambda i: (0, i, 0)),
            pl.BlockSpec((R, D), lambda i: (i, 0)),
            pl.BlockSpec((D, D), lambda i: (0, 0)),
        ],
        out_specs=pl.BlockSpec((R, D), lambda i: (i, 0)),
        out_shape=jax.ShapeDtypeStruct((N_NODES, D), jnp.float32),
    )(acc, cnt3, p, Wl)


def _tc_fc_body(h_ref, w_ref, b_ref, o_ref):
    o_ref[...] = (
        jnp.dot(h_ref[...], w_ref[...], preferred_element_type=jnp.float32)
        + b_ref[...]
    )


def _tc_fc(h2, Wfc, bfc, block_rows=1000):
    R = block_rows
    K = 5 * D
    return pl.pallas_call(
        _tc_fc_body,
        grid=(BATCH // R,),
        in_specs=[
            pl.BlockSpec((R, K), lambda i: (i, 0)),
            pl.BlockSpec((K, D), lambda i: (0, 0)),
            pl.BlockSpec((1, D), lambda i: (0, 0)),
        ],
        out_specs=pl.BlockSpec((R, D), lambda i: (i, 0)),
        out_shape=jax.ShapeDtypeStruct((BATCH, D), jnp.float32),
    )(h2, Wfc, bfc.reshape(1, D))


def kernel(x, edge_index, Wl1, bl1, Wr1, Wl2, bl2, Wr2, Wl3, bl3, Wr3,
           Wl4, bl4, Wr4, Wl5, bl5, Wr5, Wfc, bfc):
    src = edge_index[0].astype(jnp.int32)
    dst = edge_index[1].astype(jnp.int32)
    z2 = jnp.zeros((N_NODES, D), jnp.float32)

    acc, cnt = _sc_agg_cnt(x, src, dst, z2)
    p = _tc_pre(x, Wr1, bl1)
    cnt3 = cnt.reshape(NC, N_NODES, 1)
    h = _tc_post(acc, cnt3, p, Wl1)
    for Wl, bl, Wr in ((Wl2, bl2, Wr2), (Wl3, bl3, Wr3),
                       (Wl4, bl4, Wr4), (Wl5, bl5, Wr5)):
        (acc,) = _sc_agg(h, src, dst, z2)
        p = _tc_pre(h, Wr, bl)
        h = _tc_post(acc, cnt3, p, Wl)

    return _tc_fc(h.reshape(BATCH, 5 * D), Wfc, bfc)


# speedup vs baseline: 1.1573x; 1.0055x over previous
import functools

import jax
import jax.numpy as jnp
from jax import lax
from jax.experimental import pallas as pl
from jax.experimental.pallas import tpu as pltpu
from jax.experimental.pallas import tpu_sc as plsc

N_NODES = 10000
N_EDGES = 320000
D = 128
BATCH = 2000

NC = 2
NS = 16
NW = NC * NS
EPW = N_EDGES // NW
CH = 80
NCHUNK = EPW // CH
NBUF = 4
SUB_ROWS = 624
TAIL0 = NS * SUB_ROWS
TAIL_ROWS = N_NODES - TAIL0


def _sc_agg_body(with_cnt, h_hbm, src_hbm, dst_hbm, z2_hbm, *rest):
    if with_cnt:
        acc_out, cnt_out = rest[0], rest[1]
        rest = rest[2:]
    else:
        acc_out = rest[0]
        rest = rest[1:]
    srcv = rest[0:NBUF]
    dstv = rest[NBUF:2 * NBUF]
    rowsv = rest[2 * NBUF:3 * NBUF]
    onesv, zv, acc_sh, cnt_sh = rest[3 * NBUF:4 + 3 * NBUF]
    sems = rest[4 + 3 * NBUF:]
    sem_g = sems[0:NBUF]
    sem_s = sems[NBUF:2 * NBUF]
    sem_d = sems[2 * NBUF:3 * NBUF]
    sem_c = sems[3 * NBUF:4 * NBUF]
    sem_r = sems[4 * NBUF:5 * NBUF]

    c = lax.axis_index("c")
    s = lax.axis_index("s")
    r0 = pl.multiple_of(s * SUB_ROWS, 8)
    rows = pl.ds(r0, SUB_ROWS)
    tail = pl.ds(TAIL0, TAIL_ROWS)

    pltpu.sync_copy(z2_hbm.at[rows], acc_sh.at[rows])

    @pl.when(s == 0)
    def _zero_tail():
        pltpu.sync_copy(z2_hbm.at[tail], acc_sh.at[tail])

    if with_cnt:
        for j in range(SUB_ROWS // 16):
            zv[pl.ds(j * 16, 16)] = jnp.zeros((16,), jnp.float32)
        pltpu.sync_copy(zv, cnt_sh.at[rows])

        @pl.when(s == 0)
        def _zero_cnt_tail():
            pltpu.sync_copy(zv.at[pl.ds(0, TAIL_ROWS)], cnt_sh.at[tail])

        for j in range(CH // 16):
            onesv[pl.ds(j * 16, 16)] = jnp.full((16,), 1.0, jnp.float32)
    plsc.subcore_barrier()

    ebase = (c * NS + s) * EPW

    def slot(k, b):
        @pl.when(jnp.logical_and(k >= NBUF, k < NCHUNK + NBUF))
        def _wait_scatter():
            pltpu.make_async_copy(rowsv[b], acc_sh.at[dstv[b]], sem_s[b]).wait()
            if with_cnt:
                pltpu.make_async_copy(onesv, cnt_sh.at[dstv[b]], sem_c[b]).wait()

        @pl.when(k < NCHUNK)
        def _issue_idx():
            eoff = pl.multiple_of(ebase + k * CH, 8)
            pltpu.async_copy(src_hbm.at[pl.ds(eoff, CH)], srcv[b], sem_r[b])
            pltpu.async_copy(dst_hbm.at[pl.ds(eoff, CH)], dstv[b], sem_d[b])

        @pl.when(jnp.logical_and(k >= 1, k < NCHUNK + 1))
        def _gather_prev():
            p = (b + NBUF - 1) % NBUF
            pltpu.make_async_copy(
                src_hbm.at[pl.ds(0, CH)], srcv[p], sem_r[p]).wait()
            pltpu.async_copy(h_hbm.at[srcv[p]], rowsv[p], sem_g[p])

        @pl.when(jnp.logical_and(k >= 2, k < NCHUNK + 2))
        def _scatter_prev2():
            p = (b + NBUF - 2) % NBUF
            pltpu.make_async_copy(
                h_hbm.at[srcv[p]], rowsv[p], sem_g[p]).wait()
            pltpu.make_async_copy(
                dst_hbm.at[pl.ds(0, CH)], dstv[p], sem_d[p]).wait()
            pltpu.async_copy(rowsv[p], acc_sh.at[dstv[p]], sem_s[p], add=True)
            if with_cnt:
                pltpu.async_copy(onesv, cnt_sh.at[dstv[p]], sem_c[p], add=True)

    NSLOT = NCHUNK + NBUF
    NITER = (NSLOT + NBUF - 1) // NBUF

    def group(kk, carry):
        k0 = kk * NBUF
        for b in range(NBUF):
            slot(k0 + b, b)
        return carry

    lax.fori_loop(0, NITER, group, 0)
    plsc.subcore_barrier()

    pltpu.sync_copy(acc_sh.at[rows], acc_out.at[c, rows])

    @pl.when(s == 0)
    def _out_tail():
        pltpu.sync_copy(acc_sh.at[tail], acc_out.at[c, tail])

    if with_cnt:
        cbase = c * N_NODES
        pltpu.sync_copy(cnt_sh.at[rows], zv)
        pltpu.sync_copy(zv, cnt_out.at[pl.ds(pl.multiple_of(cbase + r0, 8),
                                             SUB_ROWS)])

        @pl.when(s == 0)
        def _out_cnt_tail():
            pltpu.sync_copy(cnt_sh.at[tail], zv.at[pl.ds(0, TAIL_ROWS)])
            pltpu.sync_copy(
                zv.at[pl.ds(0, TAIL_ROWS)],
                cnt_out.at[pl.ds(pl.multiple_of(cbase + TAIL0, 8), TAIL_ROWS)])


def _make_sc_agg(with_cnt):
    mesh = plsc.VectorSubcoreMesh(core_axis_name="c", subcore_axis_name="s")
    out_type = [jax.ShapeDtypeStruct((NC, N_NODES, D), jnp.float32)]
    if with_cnt:
        out_type.append(jax.ShapeDtypeStruct((NC * N_NODES,), jnp.float32))
    scratch = (
        [pltpu.VMEM((CH,), jnp.int32)] * NBUF
        + [pltpu.VMEM((CH,), jnp.int32)] * NBUF
        + [pltpu.VMEM((CH, D), jnp.float32)] * NBUF
        + [
            pltpu.VMEM((CH,), jnp.float32),
            pltpu.VMEM((SUB_ROWS,), jnp.float32),
            pltpu.VMEM_SHARED((N_NODES, D), jnp.float32),
            pltpu.VMEM_SHARED((N_NODES,), jnp.float32),
        ]
        + [pltpu.SemaphoreType.DMA] * (5 * NBUF)
    )
    return pl.kernel(
        functools.partial(_sc_agg_body, with_cnt),
        out_type=out_type,
        mesh=mesh,
        scratch_types=scratch,
    )


_sc_agg_cnt = _make_sc_agg(True)
_sc_agg = _make_sc_agg(False)


def _tc_pre_body(x_ref, wr_ref, bl_ref, o_ref):
    o_ref[...] = (
        jnp.dot(x_ref[...], wr_ref[...], preferred_element_type=jnp.float32)
        + bl_ref[...]
    )


def _tc_pre(x, Wr, bl, block_rows=2000):
    R = block_rows
    return pl.pallas_call(
        _tc_pre_body,
        grid=(N_NODES // R,),
        in_specs=[
            pl.BlockSpec((R, D), lambda i: (i, 0)),
            pl.BlockSpec((D, D), lambda i: (0, 0)),
            pl.BlockSpec((1, D), lambda i: (0, 0)),
        ],
        out_specs=pl.BlockSpec((R, D), lambda i: (i, 0)),
        out_shape=jax.ShapeDtypeStruct((N_NODES, D), jnp.float32),
    )(x, Wr, bl.reshape(1, D))


def _tc_post_body(acc_ref, cnt_ref, p_ref, wl_ref, o_ref):
    a = acc_ref[0] + acc_ref[1]
    inv = 1.0 / jnp.maximum(cnt_ref[0] + cnt_ref[1], 1.0)
    mean = a * inv
    o_ref[...] = jnp.maximum(
        jnp.dot(mean, wl_ref[...], preferred_element_type=jnp.float32)
        + p_ref[...],
        0.0,
    )


def _tc_post(acc, cnt3, p, Wl, block_rows=2000):
    R = block_rows
    return pl.pallas_call(
        _tc_post_body,
        grid=(N_NODES // R,),
        in_specs=[
            pl.BlockSpec((2, R, D), lambda i: (0, i, 0)),
            pl.BlockSpec((2, R, 1), lambda i: (0, i, 0)),
            pl.BlockSpec((R, D), lambda i: (i, 0)),
            pl.BlockSpec((D, D), lambda i: (0, 0)),
        ],
        out_specs=pl.BlockSpec((R, D), lambda i: (i, 0)),
        out_shape=jax.ShapeDtypeStruct((N_NODES, D), jnp.float32),
    )(acc, cnt3, p, Wl)


def _tc_fc_body(h_ref, w_ref, b_ref, o_ref):
    o_ref[...] = (
        jnp.dot(h_ref[...], w_ref[...], preferred_element_type=jnp.float32)
        + b_ref[...]
    )


def _tc_fc(h2, Wfc, bfc, block_rows=1000):
    R = block_rows
    K = 5 * D
    return pl.pallas_call(
        _tc_fc_body,
        grid=(BATCH // R,),
        in_specs=[
            pl.BlockSpec((R, K), lambda i: (i, 0)),
            pl.BlockSpec((K, D), lambda i: (0, 0)),
            pl.BlockSpec((1, D), lambda i: (0, 0)),
        ],
        out_specs=pl.BlockSpec((R, D), lambda i: (i, 0)),
        out_shape=jax.ShapeDtypeStruct((BATCH, D), jnp.float32),
    )(h2, Wfc, bfc.reshape(1, D))


def kernel(x, edge_index, Wl1, bl1, Wr1, Wl2, bl2, Wr2, Wl3, bl3, Wr3,
           Wl4, bl4, Wr4, Wl5, bl5, Wr5, Wfc, bfc):
    src = edge_index[0].astype(jnp.int32)
    dst = edge_index[1].astype(jnp.int32)
    z2 = jnp.zeros((N_NODES, D), jnp.float32)

    acc, cnt = _sc_agg_cnt(x, src, dst, z2)
    p = _tc_pre(x, Wr1, bl1)
    cnt3 = cnt.reshape(NC, N_NODES, 1)
    h = _tc_post(acc, cnt3, p, Wl1)
    for Wl, bl, Wr in ((Wl2, bl2, Wr2), (Wl3, bl3, Wr3),
                       (Wl4, bl4, Wr4), (Wl5, bl5, Wr5)):
        (acc,) = _sc_agg(h, src, dst, z2)
        p = _tc_pre(h, Wr, bl)
        h = _tc_post(acc, cnt3, p, Wl)

    return _tc_fc(h.reshape(BATCH, 5 * D), Wfc, bfc)
